# Initial kernel scaffold; baseline (speedup 1.0000x reference)
import jax
import jax.numpy as jnp
from jax.experimental import pallas as pl


def kernel(x, table):
    raise NotImplementedError("write your pallas kernel here")


# speedup vs baseline: 1.2567x; 1.2567x over previous
import functools
import numpy as np
import jax
import jax.numpy as jnp
from jax import lax
from jax.experimental import pallas as pl
from jax.experimental.pallas import tpu as pltpu
from jax.experimental.pallas import tpu_sc as plsc

DIM = 32
SCALE = np.sqrt(np.float32(DIM)).astype(np.float32)
NC, NS = 2, 16
NW = NC * NS
CHUNK = 512
KSTREAM = CHUNK // 128


@functools.lru_cache(maxsize=None)
def _make(B):
    n_chunks = B // (NW * CHUNK)
    mesh = plsc.VectorSubcoreMesh(
        core_axis_name="c", subcore_axis_name="s",
        num_cores=NC, num_subcores=NS)

    @functools.partial(
        pl.kernel,
        out_type=jax.ShapeDtypeStruct((B, DIM), jnp.float32),
        mesh=mesh,
        scratch_types=[
            pltpu.VMEM((KSTREAM, 128), jnp.int32),
            pltpu.VMEM((CHUNK, DIM), jnp.float32),
            pltpu.SemaphoreType.DMA,
        ],
        compiler_params=pltpu.CompilerParams(use_tc_tiling_on_sc=False),
    )
    def emb_kernel(table_hbm, idx_hbm, out_hbm, idx_v, buf, sem):
        wid = lax.axis_index("s") * NC + lax.axis_index("c")
        row0 = wid * (n_chunks * CHUNK)

        @pl.loop(0, n_chunks)
        def _chunk(c):
            pltpu.sync_copy(idx_hbm.at[wid, c], idx_v)
            cps = [
                pltpu.async_copy(
                    table_hbm.at[idx_v.at[j]],
                    buf.at[pl.ds(j * 128, 128)],
                    sem,
                )
                for j in range(KSTREAM)
            ]
            for cp in cps:
                cp.wait()

            @pl.loop(0, CHUNK)
            def _scale(r):
                buf[r, pl.ds(0, 16)] = buf[r, pl.ds(0, 16)] * SCALE
                buf[r, pl.ds(16, 16)] = buf[r, pl.ds(16, 16)] * SCALE

            pltpu.sync_copy(buf, out_hbm.at[pl.ds(row0 + c * CHUNK, CHUNK)])

    return emb_kernel


def kernel(x, table):
    B = x.size
    idx = x.reshape(NW, B // (NW * CHUNK), KSTREAM, 128).astype(jnp.int32)
    out = _make(B)(table, idx)
    return out.reshape(x.shape + (DIM,))


# speedup vs baseline: 1.4806x; 1.1781x over previous
import functools
import numpy as np
import jax
import jax.numpy as jnp
from jax import lax
from jax.experimental import pallas as pl
from jax.experimental.pallas import tpu as pltpu
from jax.experimental.pallas import tpu_sc as plsc

DIM = 32
SCALE = np.sqrt(np.float32(DIM)).astype(np.float32)
NC, NS = 2, 16
NW = NC * NS
CHUNK = 512
KSTREAM = CHUNK // 128
NBUF = 5
GATHER_AHEAD = 3


@functools.lru_cache(maxsize=None)
def _make(B):
    b_per_w = B // NW
    n_chunks = b_per_w // CHUNK
    n_groups = n_chunks // NBUF
    assert n_chunks % NBUF == 0 and n_chunks > NBUF
    mesh = plsc.VectorSubcoreMesh(
        core_axis_name="c", subcore_axis_name="s",
        num_cores=NC, num_subcores=NS)

    @functools.partial(
        pl.kernel,
        out_type=jax.ShapeDtypeStruct((B, DIM), jnp.float32),
        mesh=mesh,
        scratch_types=(
            [pltpu.VMEM((n_chunks * KSTREAM, 128), jnp.int32)]
            + [pltpu.VMEM((CHUNK, DIM), jnp.float32)] * NBUF
            + [pltpu.SemaphoreType.DMA] * (2 * NBUF)
        ),
        compiler_params=pltpu.CompilerParams(use_tc_tiling_on_sc=False),
    )
    def emb_kernel(table_hbm, idx_hbm, out_hbm, idx_v, *scratch):
        bufs = scratch[:NBUF]
        gsems = scratch[NBUF:2 * NBUF]
        ssems = scratch[2 * NBUF:]
        wid = lax.axis_index("s") * NC + lax.axis_index("c")
        row0 = wid * b_per_w

        def fire_gather(c, b):
            for j in range(KSTREAM):
                pltpu.async_copy(
                    table_hbm.at[idx_v.at[c * KSTREAM + j]],
                    bufs[b].at[pl.ds(j * 128, 128)],
                    gsems[b])

        def wait_gather(b):
            pltpu.make_async_copy(
                out_hbm.at[pl.ds(0, CHUNK)], bufs[b], gsems[b]).wait()

        def fire_scatter(c, b):
            pltpu.async_copy(
                bufs[b], out_hbm.at[pl.ds(row0 + c * CHUNK, CHUNK)], ssems[b])

        def wait_scatter(b):
            pltpu.make_async_copy(
                bufs[b], out_hbm.at[pl.ds(0, CHUNK)], ssems[b]).wait()

        def scale(b):
            buf = bufs[b]

            @plsc.parallel_loop(0, CHUNK, step=1, unroll=8)
            def _scale(r):
                buf[r, pl.ds(0, 16)] = buf[r, pl.ds(0, 16)] * SCALE
                buf[r, pl.ds(16, 16)] = buf[r, pl.ds(16, 16)] * SCALE

        pltpu.sync_copy(idx_hbm.at[wid], idx_v)

        for c in range(GATHER_AHEAD):
            fire_gather(c, c % NBUF)

        @pl.loop(0, n_groups)
        def _group(g):
            for i in range(NBUF):
                c = g * NBUF + i
                wait_gather(i)
                scale(i)
                fire_scatter(c, i)
                inext = (i + GATHER_AHEAD) % NBUF

                @pl.when(c + GATHER_AHEAD < n_chunks)
                def _():
                    @pl.when(c + GATHER_AHEAD >= NBUF)
                    def _():
                        wait_scatter(inext)
                    fire_gather(c + GATHER_AHEAD, inext)

        for c in range(n_chunks - NBUF, n_chunks):
            wait_scatter(c % NBUF)

    return emb_kernel


def kernel(x, table):
    B = x.size
    idx = x.reshape(NW, (B // NW) // 128, 128).astype(jnp.int32)
    out = _make(B)(table, idx)
    return out.reshape(x.shape + (DIM,))
